# Initial kernel scaffold; baseline (speedup 1.0000x reference)
import functools

import jax
import jax.numpy as jnp
from jax import lax
from jax.experimental import pallas as pl
from jax.experimental.pallas import tpu as pltpu
from jax.experimental.pallas import tpu_sc as plsc

D = 64

try:
    _info = plsc.get_sparse_core_info()
    _NC, _NS = _info.num_cores, _info.num_subcores
except Exception:
    _NC, _NS = 2, 16
_NW = _NC * _NS


def _pad_to(n: int, mult: int) -> int:
    return ((n + mult - 1) // mult) * mult


def _sc_gather(cemb, memb, emb0, emb1, cit_p, mit_p):
    dpad = cit_p.shape[0]
    mpad = mit_p.shape[0]
    bd = dpad // _NW
    bm = mpad // _NW
    mesh = plsc.VectorSubcoreMesh(core_axis_name="c", subcore_axis_name="s")

    @functools.partial(
        pl.kernel,
        mesh=mesh,
        out_type=[
            jax.ShapeDtypeStruct((dpad, D), jnp.float32),
            jax.ShapeDtypeStruct((mpad, D), jnp.float32),
            jax.ShapeDtypeStruct((dpad, D), jnp.float32),
            jax.ShapeDtypeStruct((mpad, D), jnp.float32),
        ],
        scratch_types=[
            pltpu.VMEM((bd,), jnp.int32),
            pltpu.VMEM((bm,), jnp.int32),
            pltpu.VMEM((bd, D), jnp.float32),
            pltpu.VMEM((bd, D), jnp.float32),
            pltpu.VMEM((bm, D), jnp.float32),
            pltpu.VMEM((bm, D), jnp.float32),
            pltpu.SemaphoreType.DMA,
            pltpu.SemaphoreType.DMA,
            pltpu.SemaphoreType.DMA,
            pltpu.SemaphoreType.DMA,
        ],
    )
    def k(cemb_h, memb_h, e0_h, e1_h, cit_h, mit_h,
          dia_o, med_o, g0_o, g1_o,
          idxd_v, idxm_v, rd1, rd2, rm1, rm2, s1, s2, s3, s4):
        wid = lax.axis_index("s") * _NC + lax.axis_index("c")
        od = wid * bd
        om = wid * bm
        pltpu.sync_copy(cit_h.at[pl.ds(od, bd)], idxd_v)
        pltpu.sync_copy(mit_h.at[pl.ds(om, bm)], idxm_v)
        c1 = pltpu.async_copy(cemb_h.at[idxd_v], rd1, s1)
        c2 = pltpu.async_copy(e0_h.at[idxd_v], rd2, s2)
        c3 = pltpu.async_copy(memb_h.at[idxm_v], rm1, s3)
        c4 = pltpu.async_copy(e1_h.at[idxm_v], rm2, s4)
        c1.wait()
        pltpu.sync_copy(rd1, dia_o.at[pl.ds(od, bd)])
        c2.wait()
        pltpu.sync_copy(rd2, g0_o.at[pl.ds(od, bd)])
        c3.wait()
        pltpu.sync_copy(rm1, med_o.at[pl.ds(om, bm)])
        c4.wait()
        pltpu.sync_copy(rm2, g1_o.at[pl.ds(om, bm)])

    return k(cemb, memb, emb0, emb1, cit_p, mit_p)


def _tc_body(nd, nm, dia_ref, med_ref, e0_ref, e1_ref, hea_ref,
             w1_ref, w2_ref, wl_ref, b1_ref, b2_ref, att_ref,
             o1_ref, o2_ref):
    f32 = jnp.float32
    hi = lax.Precision.HIGHEST

    def dot(a, b, dn):
        return lax.dot_general(a, b, dimension_numbers=(dn, ((), ())),
                               preferred_element_type=f32, precision=hi)

    dia = dia_ref[:]
    med = med_ref[:]
    hea = hea_ref[:]
    W1 = w1_ref[:]
    W2 = w2_ref[:]
    Wl = wl_ref[:]
    b1 = b1_ref[:]
    b2 = b2_ref[:]
    att = att_ref[:]
    att1 = att[:, :D]
    att2 = att[:, D:]

    dpad = dia.shape[0]
    mpad = med.shape[0]
    mask_d = lax.broadcasted_iota(jnp.int32, (dpad, 1), 0) < nd
    mask_m = lax.broadcasted_iota(jnp.int32, (mpad, 1), 0) < nm
    cmask = lax.broadcasted_iota(jnp.int32, (1, mpad), 1) < nm

    xw_d = dot(dia[:nd], W2, ((1,), (0,)))
    xw_m = dot(med, W2, ((1,), (0,)))
    ew = dot(hea, W2, ((1,), (0,)))

    a_d = dot(xw_d, att1, ((1,), (1,)))
    b_e = dot(ew, att2, ((1,), (1,)))
    a_m = dot(att1, xw_m, ((1,), (1,)))

    def lrelu(x):
        return jnp.where(x > 0, x, 0.2 * x)

    s = jnp.where(cmask, lrelu(a_m + b_e), f32(-1e30))
    t = lrelu(a_d + b_e)
    mx = jnp.maximum(jnp.max(s, axis=1, keepdims=True), t)
    e = jnp.exp(s - mx)
    ed = jnp.exp(t - mx)
    se = jnp.sum(e, axis=1, keepdims=True)
    denom = ed + se + f32(1e-16)
    alpha_dd = ed / denom
    exm = dot(e, xw_m, ((1,), (0,)))
    ef = (alpha_dd * xw_d + exm / denom) * (f32(1.0) / f32(nm + 1))
    w = se / denom
    s_dis = jnp.sum(alpha_dd * ef, axis=0, keepdims=True)
    s_med = jnp.sum(w * ef, axis=0, keepdims=True) * (f32(1.0) / f32(nd))

    sd = jnp.sum(jnp.where(mask_d, dia, f32(0.0)), axis=0, keepdims=True)
    sm = jnp.sum(jnp.where(mask_m, med, f32(0.0)), axis=0, keepdims=True)
    se0 = jnp.sum(jnp.where(mask_d, e0_ref[:], f32(0.0)), axis=0, keepdims=True)
    se1 = jnp.sum(jnp.where(mask_m, e1_ref[:], f32(0.0)), axis=0, keepdims=True)

    sum_dia_feat = dot(sd, W1, ((1,), (0,))) + f32(nd) * b1
    sum_med_feat = dot(sm, W1, ((1,), (0,))) + f32(nm) * b1
    u1 = jnp.concatenate([s_dis + f32(nd) * b2, sum_dia_feat], axis=1)
    u2 = jnp.concatenate([s_med + f32(nm) * b2, sum_med_feat], axis=1)
    o1_ref[:] = se0 + dot(u1, Wl, ((1,), (1,)))
    o2_ref[:] = se1 + dot(u2, Wl, ((1,), (1,)))


def _tc_call(nd, nm, dia_g, med_g, e0_g, e1_g, hyperedge_attr,
             W1, W2, Wl, b1, b2, att, interpret=False):
    return pl.pallas_call(
        functools.partial(_tc_body, nd, nm),
        out_shape=[jax.ShapeDtypeStruct((1, D), jnp.float32)] * 2,
        interpret=interpret,
    )(dia_g, med_g, e0_g, e1_g, hyperedge_attr, W1, W2, Wl,
      b1.reshape(1, D), b2.reshape(1, D), att.reshape(1, 2 * D))


def kernel(c_embeddings, m_embeddings, emb0, emb1, W1, b1, W2, b2, att, Wl,
           hyperedge_attr, c_it, medicine_it):
    nd = c_it.shape[0]
    nm = medicine_it.shape[0]
    dpad = _pad_to(nd, 8 * _NW)
    mpad = _pad_to(nm, 8 * _NW)
    cit_p = jnp.concatenate(
        [c_it.astype(jnp.int32), jnp.zeros((dpad - nd,), jnp.int32)])
    mit_p = jnp.concatenate(
        [medicine_it.astype(jnp.int32), jnp.zeros((mpad - nm,), jnp.int32)])
    dia_g, med_g, e0_g, e1_g = _sc_gather(
        c_embeddings, m_embeddings, emb0, emb1, cit_p, mit_p)
    o1, o2 = _tc_call(nd, nm, dia_g, med_g, e0_g, e1_g, hyperedge_attr,
                      W1, W2, Wl, b1, b2, att)
    return (o1.reshape(1, 1, D), o2.reshape(1, 1, D))


# speedup vs baseline: 100.2813x; 100.2813x over previous
import functools

import jax
import jax.numpy as jnp
from jax import lax
from jax.experimental import pallas as pl
from jax.experimental.pallas import tpu as pltpu
from jax.experimental.pallas import tpu_sc as plsc

D = 64

try:
    _info = plsc.get_sparse_core_info()
    _NC, _NS = _info.num_cores, _info.num_subcores
except Exception:
    _NC, _NS = 2, 16
_NW = _NC * _NS


def _pad_to(n: int, mult: int) -> int:
    return ((n + mult - 1) // mult) * mult


def _sc_gather(cemb, memb, emb0, emb1, cit_p, mit_p):
    dpad = cit_p.shape[0]
    mpad = mit_p.shape[0]
    bd = dpad // _NW
    bm = mpad // _NW
    mesh = plsc.VectorSubcoreMesh(core_axis_name="c", subcore_axis_name="s")

    @functools.partial(
        pl.kernel,
        mesh=mesh,
        compiler_params=pltpu.CompilerParams(use_tc_tiling_on_sc=False),
        out_type=[
            jax.ShapeDtypeStruct((dpad, D), jnp.float32),
            jax.ShapeDtypeStruct((mpad, D), jnp.float32),
            jax.ShapeDtypeStruct((dpad, D), jnp.float32),
            jax.ShapeDtypeStruct((mpad, D), jnp.float32),
        ],
        scratch_types=[
            pltpu.VMEM((bd,), jnp.int32),
            pltpu.VMEM((bm,), jnp.int32),
            pltpu.VMEM((bd, D), jnp.float32),
            pltpu.VMEM((bd, D), jnp.float32),
            pltpu.VMEM((bm, D), jnp.float32),
            pltpu.VMEM((bm, D), jnp.float32),
            pltpu.SemaphoreType.DMA,
            pltpu.SemaphoreType.DMA,
            pltpu.SemaphoreType.DMA,
            pltpu.SemaphoreType.DMA,
        ],
    )
    def k(cemb_h, memb_h, e0_h, e1_h, cit_h, mit_h,
          dia_o, med_o, g0_o, g1_o,
          idxd_v, idxm_v, rd1, rd2, rm1, rm2, s1, s2, s3, s4):
        wid = lax.axis_index("s") * _NC + lax.axis_index("c")
        od = wid * bd
        om = wid * bm
        pltpu.sync_copy(cit_h.at[pl.ds(od, bd)], idxd_v)
        pltpu.sync_copy(mit_h.at[pl.ds(om, bm)], idxm_v)
        c1 = pltpu.async_copy(cemb_h.at[idxd_v], rd1, s1)
        c2 = pltpu.async_copy(e0_h.at[idxd_v], rd2, s2)
        c3 = pltpu.async_copy(memb_h.at[idxm_v], rm1, s3)
        c4 = pltpu.async_copy(e1_h.at[idxm_v], rm2, s4)
        c1.wait()
        pltpu.sync_copy(rd1, dia_o.at[pl.ds(od, bd)])
        c2.wait()
        pltpu.sync_copy(rd2, g0_o.at[pl.ds(od, bd)])
        c3.wait()
        pltpu.sync_copy(rm1, med_o.at[pl.ds(om, bm)])
        c4.wait()
        pltpu.sync_copy(rm2, g1_o.at[pl.ds(om, bm)])

    return k(cemb, memb, emb0, emb1, cit_p, mit_p)


def _tc_body(nd, nm, dia_ref, med_ref, e0_ref, e1_ref, hea_ref,
             w1_ref, w2_ref, wl_ref, b1_ref, b2_ref, att_ref,
             o1_ref, o2_ref):
    f32 = jnp.float32
    hi = lax.Precision.HIGHEST

    def dot(a, b, dn):
        return lax.dot_general(a, b, dimension_numbers=(dn, ((), ())),
                               preferred_element_type=f32, precision=hi)

    dia = dia_ref[:]
    med = med_ref[:]
    hea = hea_ref[:]
    W1 = w1_ref[:]
    W2 = w2_ref[:]
    Wl = wl_ref[:]
    b1 = b1_ref[:]
    b2 = b2_ref[:]
    att = att_ref[:]
    att1 = att[:, :D]
    att2 = att[:, D:]

    dpad = dia.shape[0]
    mpad = med.shape[0]
    mask_d = lax.broadcasted_iota(jnp.int32, (dpad, 1), 0) < nd
    mask_m = lax.broadcasted_iota(jnp.int32, (mpad, 1), 0) < nm
    cmask = lax.broadcasted_iota(jnp.int32, (1, mpad), 1) < nm

    xw_d = dot(dia[:nd], W2, ((1,), (0,)))
    xw_m = dot(med, W2, ((1,), (0,)))
    ew = dot(hea, W2, ((1,), (0,)))

    a_d = dot(xw_d, att1, ((1,), (1,)))
    b_e = dot(ew, att2, ((1,), (1,)))
    a_m = dot(att1, xw_m, ((1,), (1,)))

    def lrelu(x):
        return jnp.where(x > 0, x, 0.2 * x)

    s = jnp.where(cmask, lrelu(a_m + b_e), f32(-1e30))
    t = lrelu(a_d + b_e)
    mx = jnp.maximum(jnp.max(s, axis=1, keepdims=True), t)
    e = jnp.exp(s - mx)
    ed = jnp.exp(t - mx)
    se = jnp.sum(e, axis=1, keepdims=True)
    denom = ed + se + f32(1e-16)
    alpha_dd = ed / denom
    exm = dot(e, xw_m, ((1,), (0,)))
    ef = (alpha_dd * xw_d + exm / denom) * (f32(1.0) / f32(nm + 1))
    w = se / denom
    s_dis = jnp.sum(alpha_dd * ef, axis=0, keepdims=True)
    s_med = jnp.sum(w * ef, axis=0, keepdims=True) * (f32(1.0) / f32(nd))

    sd = jnp.sum(jnp.where(mask_d, dia, f32(0.0)), axis=0, keepdims=True)
    sm = jnp.sum(jnp.where(mask_m, med, f32(0.0)), axis=0, keepdims=True)
    se0 = jnp.sum(jnp.where(mask_d, e0_ref[:], f32(0.0)), axis=0, keepdims=True)
    se1 = jnp.sum(jnp.where(mask_m, e1_ref[:], f32(0.0)), axis=0, keepdims=True)

    sum_dia_feat = dot(sd, W1, ((1,), (0,))) + f32(nd) * b1
    sum_med_feat = dot(sm, W1, ((1,), (0,))) + f32(nm) * b1
    u1 = jnp.concatenate([s_dis + f32(nd) * b2, sum_dia_feat], axis=1)
    u2 = jnp.concatenate([s_med + f32(nm) * b2, sum_med_feat], axis=1)
    o1_ref[:] = se0 + dot(u1, Wl, ((1,), (1,)))
    o2_ref[:] = se1 + dot(u2, Wl, ((1,), (1,)))


def _tc_call(nd, nm, dia_g, med_g, e0_g, e1_g, hyperedge_attr,
             W1, W2, Wl, b1, b2, att, interpret=False):
    return pl.pallas_call(
        functools.partial(_tc_body, nd, nm),
        out_shape=[jax.ShapeDtypeStruct((1, D), jnp.float32)] * 2,
        interpret=interpret,
    )(dia_g, med_g, e0_g, e1_g, hyperedge_attr, W1, W2, Wl,
      b1.reshape(1, D), b2.reshape(1, D), att.reshape(1, 2 * D))


def kernel(c_embeddings, m_embeddings, emb0, emb1, W1, b1, W2, b2, att, Wl,
           hyperedge_attr, c_it, medicine_it):
    nd = c_it.shape[0]
    nm = medicine_it.shape[0]
    dpad = _pad_to(nd, 8 * _NW)
    mpad = _pad_to(nm, 8 * _NW)
    cit_p = jnp.concatenate(
        [c_it.astype(jnp.int32), jnp.zeros((dpad - nd,), jnp.int32)])
    mit_p = jnp.concatenate(
        [medicine_it.astype(jnp.int32), jnp.zeros((mpad - nm,), jnp.int32)])
    dia_g, med_g, e0_g, e1_g = _sc_gather(
        c_embeddings, m_embeddings, emb0, emb1, cit_p, mit_p)
    o1, o2 = _tc_call(nd, nm, dia_g, med_g, e0_g, e1_g, hyperedge_attr,
                      W1, W2, Wl, b1, b2, att)
    return (o1.reshape(1, 1, D), o2.reshape(1, 1, D))


# speedup vs baseline: 117.9102x; 1.1758x over previous
import functools

import jax
import jax.numpy as jnp
from jax import lax
from jax.experimental import pallas as pl
from jax.experimental.pallas import tpu as pltpu
from jax.experimental.pallas import tpu_sc as plsc

D = 64

try:
    _info = plsc.get_sparse_core_info()
    _NC, _NS = _info.num_cores, _info.num_subcores
except Exception:
    _NC, _NS = 2, 16
_NW = _NC * _NS


def _pad_to(n: int, mult: int) -> int:
    return ((n + mult - 1) // mult) * mult


def _sc_gather(cemb, memb, emb0, emb1, cit_p, mit_p):
    dpad = cit_p.shape[0]
    mpad = mit_p.shape[0]
    bd = dpad // _NW
    bm = mpad // _NW
    mesh = plsc.VectorSubcoreMesh(core_axis_name="c", subcore_axis_name="s")

    @functools.partial(
        pl.kernel,
        mesh=mesh,
        out_type=[
            jax.ShapeDtypeStruct((dpad, D), jnp.float32),
            jax.ShapeDtypeStruct((mpad, D), jnp.float32),
            jax.ShapeDtypeStruct((dpad, D), jnp.float32),
            jax.ShapeDtypeStruct((mpad, D), jnp.float32),
        ],
        scratch_types=[
            pltpu.VMEM((bd,), jnp.int32),
            pltpu.VMEM((bm,), jnp.int32),
            pltpu.SemaphoreType.DMA,
        ],
    )
    def k(cemb_h, memb_h, e0_h, e1_h, cit_h, mit_h,
          dia_o, med_o, g0_o, g1_o,
          idxd_s, idxm_s, sem):
        wid = lax.axis_index("s") * _NC + lax.axis_index("c")
        od = wid * bd
        om = wid * bm
        pltpu.sync_copy(cit_h.at[pl.ds(od, bd)], idxd_s)
        pltpu.sync_copy(mit_h.at[pl.ds(om, bm)], idxm_s)
        vecs_d = [idxd_s[pl.ds(16 * t, 16)] for t in range(bd // 16)]
        vecs_m = [idxm_s[pl.ds(16 * t, 16)] for t in range(bm // 16)]
        descs = []
        for i in range(bd):
            s = vecs_d[i // 16][i % 16]
            descs.append(pltpu.make_async_copy(
                cemb_h.at[pl.ds(s, 1)], dia_o.at[pl.ds(od + i, 1)], sem))
            descs.append(pltpu.make_async_copy(
                e0_h.at[pl.ds(s, 1)], g0_o.at[pl.ds(od + i, 1)], sem))
        for i in range(bm):
            s = vecs_m[i // 16][i % 16]
            descs.append(pltpu.make_async_copy(
                memb_h.at[pl.ds(s, 1)], med_o.at[pl.ds(om + i, 1)], sem))
            descs.append(pltpu.make_async_copy(
                e1_h.at[pl.ds(s, 1)], g1_o.at[pl.ds(om + i, 1)], sem))
        for d in descs:
            d.start()
        for d in descs:
            d.wait()

    return k(cemb, memb, emb0, emb1, cit_p, mit_p)


def _tc_body(nd, nm, dia_ref, med_ref, e0_ref, e1_ref, hea_ref,
             w1_ref, w2_ref, wl_ref, b1_ref, b2_ref, att_ref,
             o1_ref, o2_ref):
    f32 = jnp.float32
    hi = lax.Precision.HIGHEST

    def dot(a, b, dn):
        return lax.dot_general(a, b, dimension_numbers=(dn, ((), ())),
                               preferred_element_type=f32, precision=hi)

    dia = dia_ref[:]
    med = med_ref[:]
    hea = hea_ref[:]
    W1 = w1_ref[:]
    W2 = w2_ref[:]
    Wl = wl_ref[:]
    b1 = b1_ref[:]
    b2 = b2_ref[:]
    att = att_ref[:]
    att1 = att[:, :D]
    att2 = att[:, D:]

    dpad = dia.shape[0]
    mpad = med.shape[0]
    mask_d = lax.broadcasted_iota(jnp.int32, (dpad, 1), 0) < nd
    mask_m = lax.broadcasted_iota(jnp.int32, (mpad, 1), 0) < nm
    cmask = lax.broadcasted_iota(jnp.int32, (1, mpad), 1) < nm

    xw_d = dot(dia[:nd], W2, ((1,), (0,)))
    xw_m = dot(med, W2, ((1,), (0,)))
    ew = dot(hea, W2, ((1,), (0,)))

    a_d = dot(xw_d, att1, ((1,), (1,)))
    b_e = dot(ew, att2, ((1,), (1,)))
    a_m = dot(att1, xw_m, ((1,), (1,)))

    def lrelu(x):
        return jnp.where(x > 0, x, 0.2 * x)

    s = jnp.where(cmask, lrelu(a_m + b_e), f32(-1e30))
    t = lrelu(a_d + b_e)
    mx = jnp.maximum(jnp.max(s, axis=1, keepdims=True), t)
    e = jnp.exp(s - mx)
    ed = jnp.exp(t - mx)
    se = jnp.sum(e, axis=1, keepdims=True)
    denom = ed + se + f32(1e-16)
    alpha_dd = ed / denom
    exm = dot(e, xw_m, ((1,), (0,)))
    ef = (alpha_dd * xw_d + exm / denom) * (f32(1.0) / f32(nm + 1))
    w = se / denom
    s_dis = jnp.sum(alpha_dd * ef, axis=0, keepdims=True)
    s_med = jnp.sum(w * ef, axis=0, keepdims=True) * (f32(1.0) / f32(nd))

    sd = jnp.sum(jnp.where(mask_d, dia, f32(0.0)), axis=0, keepdims=True)
    sm = jnp.sum(jnp.where(mask_m, med, f32(0.0)), axis=0, keepdims=True)
    se0 = jnp.sum(jnp.where(mask_d, e0_ref[:], f32(0.0)), axis=0, keepdims=True)
    se1 = jnp.sum(jnp.where(mask_m, e1_ref[:], f32(0.0)), axis=0, keepdims=True)

    sum_dia_feat = dot(sd, W1, ((1,), (0,))) + f32(nd) * b1
    sum_med_feat = dot(sm, W1, ((1,), (0,))) + f32(nm) * b1
    u1 = jnp.concatenate([s_dis + f32(nd) * b2, sum_dia_feat], axis=1)
    u2 = jnp.concatenate([s_med + f32(nm) * b2, sum_med_feat], axis=1)
    o1_ref[:] = se0 + dot(u1, Wl, ((1,), (1,)))
    o2_ref[:] = se1 + dot(u2, Wl, ((1,), (1,)))


def _tc_call(nd, nm, dia_g, med_g, e0_g, e1_g, hyperedge_attr,
             W1, W2, Wl, b1, b2, att, interpret=False):
    return pl.pallas_call(
        functools.partial(_tc_body, nd, nm),
        out_shape=[jax.ShapeDtypeStruct((1, D), jnp.float32)] * 2,
        interpret=interpret,
    )(dia_g, med_g, e0_g, e1_g, hyperedge_attr, W1, W2, Wl,
      b1.reshape(1, D), b2.reshape(1, D), att.reshape(1, 2 * D))


def kernel(c_embeddings, m_embeddings, emb0, emb1, W1, b1, W2, b2, att, Wl,
           hyperedge_attr, c_it, medicine_it):
    nd = c_it.shape[0]
    nm = medicine_it.shape[0]
    dpad = _pad_to(nd, 16 * _NW)
    mpad = _pad_to(nm, 16 * _NW)
    cit_p = jnp.concatenate(
        [c_it.astype(jnp.int32), jnp.zeros((dpad - nd,), jnp.int32)])
    mit_p = jnp.concatenate(
        [medicine_it.astype(jnp.int32), jnp.zeros((mpad - nm,), jnp.int32)])
    dia_g, med_g, e0_g, e1_g = _sc_gather(
        c_embeddings, m_embeddings, emb0, emb1, cit_p, mit_p)
    o1, o2 = _tc_call(nd, nm, dia_g, med_g, e0_g, e1_g, hyperedge_attr,
                      W1, W2, Wl, b1, b2, att)
    return (o1.reshape(1, 1, D), o2.reshape(1, 1, D))


# speedup vs baseline: 336.0208x; 2.8498x over previous
import functools

import jax
import jax.numpy as jnp
from jax import lax
from jax.experimental import pallas as pl
from jax.experimental.pallas import tpu as pltpu
from jax.experimental.pallas import tpu_sc as plsc

D = 64
LANES = 128
NBUF = 4

try:
    _info = plsc.get_sparse_core_info()
    _NC, _NS = _info.num_cores, _info.num_subcores
except Exception:
    _NC, _NS = 2, 16
_NW = _NC * _NS


def _pad_to(n: int, mult: int) -> int:
    return ((n + mult - 1) // mult) * mult


def _sc_gather(cemb_t, memb_t, emb0_t, emb1_t, cit_p, mit_p, nm):
    dpad = cit_p.shape[0]
    mpad = mit_p.shape[0]
    bd = dpad // _NW
    bm = mpad // _NW
    mesh = plsc.VectorSubcoreMesh(core_axis_name="c", subcore_axis_name="s")

    @functools.partial(
        pl.kernel,
        mesh=mesh,
        compiler_params=pltpu.CompilerParams(needs_layout_passes=False),
        out_type=[
            jax.ShapeDtypeStruct((dpad, D), jnp.float32),
            jax.ShapeDtypeStruct((mpad, D), jnp.float32),
            jax.ShapeDtypeStruct((dpad, D), jnp.float32),
            jax.ShapeDtypeStruct((mpad, D), jnp.float32),
        ],
        scratch_types=[
            pltpu.VMEM((bd,), jnp.int32),
            pltpu.VMEM((bm,), jnp.int32),
            pltpu.VMEM((bd, D), jnp.float32),
            pltpu.VMEM((bm, D), jnp.float32),
            pltpu.VMEM((bd, D), jnp.float32),
            pltpu.VMEM((bm, D), jnp.float32),
            [pltpu.VMEM((D, LANES), jnp.float32) for _ in range(NBUF)],
            [pltpu.SemaphoreType.DMA for _ in range(NBUF)],
        ],
    )
    def k(cemb_h, memb_h, e0_h, e1_h, cit_h, mit_h,
          dia_o, med_o, g0_o, g1_o,
          idxd_v, idxm_v, st_dia, st_med, st_e0, st_e1, blks, sems):
        wid = lax.axis_index("s") * _NC + lax.axis_index("c")
        od = wid * bd
        om = wid * bm
        pltpu.sync_copy(cit_h.at[pl.ds(od, bd)], idxd_v)
        pltpu.sync_copy(mit_h.at[pl.ds(om, bm)], idxm_v)
        vecs_d = [idxd_v[pl.ds(16 * t, 16)] for t in range(bd // 16)]
        vecs_m = [idxm_v[pl.ds(16 * t, 16)] for t in range(bm // 16)]

        jobs = []
        for i in range(bd):
            s = vecs_d[i // 16][i % 16]
            jobs.append((cemb_h, st_dia, i, s, None))
            jobs.append((e0_h, st_e0, i, s, None))
        for i in range(bm):
            s = vecs_m[i // 16][i % 16]
            cond = om + i < nm
            jobs.append((memb_h, st_med, i, s, cond))
            jobs.append((e1_h, st_e1, i, s, cond))

        rows_q = [lax.iota(jnp.int32, 16) + 16 * q for q in range(D // 16)]

        def fire(j, slot):
            src, _, _, s, cond = jobs[j]
            start = pl.multiple_of((s // LANES) * LANES, LANES)

            def do():
                pltpu.make_async_copy(
                    src.at[:, pl.ds(start, LANES)], blks[slot],
                    sems[slot]).start()
            if cond is None:
                do()
            else:
                pl.when(cond)(do)

        def drain(j, slot):
            src, stage, row, s, cond = jobs[j]
            lo = s % LANES

            def do():
                pltpu.make_async_copy(
                    src.at[:, pl.ds(0, LANES)], blks[slot],
                    sems[slot]).wait()
                colv = jnp.full((16,), lo, jnp.int32)
                for q in range(D // 16):
                    g = plsc.load_gather(blks[slot], [rows_q[q], colv])
                    stage[row, pl.ds(16 * q, 16)] = g
            if cond is None:
                do()
            else:
                pl.when(cond)(do)

        n = len(jobs)
        for j in range(n + NBUF):
            if j >= NBUF:
                drain(j - NBUF, (j - NBUF) % NBUF)
            if j < n:
                fire(j, j % NBUF)

        pltpu.sync_copy(st_dia, dia_o.at[pl.ds(od, bd)])
        pltpu.sync_copy(st_e0, g0_o.at[pl.ds(od, bd)])
        pltpu.sync_copy(st_med, med_o.at[pl.ds(om, bm)])
        pltpu.sync_copy(st_e1, g1_o.at[pl.ds(om, bm)])

    return k(cemb_t, memb_t, emb0_t, emb1_t, cit_p, mit_p)


def _tc_body(nd, nm, dia_ref, med_ref, e0_ref, e1_ref, hea_ref,
             w1_ref, w2_ref, wl_ref, b1_ref, b2_ref, att_ref,
             o1_ref, o2_ref):
    f32 = jnp.float32
    hi = lax.Precision.HIGHEST

    def dot(a, b, dn):
        return lax.dot_general(a, b, dimension_numbers=(dn, ((), ())),
                               preferred_element_type=f32, precision=hi)

    dia = dia_ref[:]
    hea_t = hea_ref[:]
    W1 = w1_ref[:]
    W2 = w2_ref[:]
    Wl = wl_ref[:]
    b1 = b1_ref[:]
    b2 = b2_ref[:]
    att = att_ref[:]
    att1 = att[:, :D]
    att2 = att[:, D:]

    dpad = dia.shape[0]
    mpad = med_ref.shape[0]
    mask_d = lax.broadcasted_iota(jnp.int32, (dpad, 1), 0) < nd
    mask_m = lax.broadcasted_iota(jnp.int32, (mpad, 1), 0) < nm
    cmask = lax.broadcasted_iota(jnp.int32, (1, mpad), 1) < nm

    med = jnp.where(mask_m, med_ref[:], f32(0.0))

    xw_d = dot(dia[:nd], W2, ((1,), (0,)))
    xw_m = dot(med, W2, ((1,), (0,)))
    ew = dot(hea_t, W2, ((0,), (0,)))

    a_d = dot(xw_d, att1, ((1,), (1,)))
    b_e = dot(ew, att2, ((1,), (1,)))
    a_m = dot(att1, xw_m, ((1,), (1,)))

    def lrelu(x):
        return jnp.where(x > 0, x, 0.2 * x)

    s = jnp.where(cmask, lrelu(a_m + b_e), f32(-1e30))
    t = lrelu(a_d + b_e)
    mx = jnp.maximum(jnp.max(s, axis=1, keepdims=True), t)
    e = jnp.exp(s - mx)
    ed = jnp.exp(t - mx)
    se = jnp.sum(e, axis=1, keepdims=True)
    denom = ed + se + f32(1e-16)
    alpha_dd = ed / denom
    exm = dot(e, xw_m, ((1,), (0,)))
    ef = (alpha_dd * xw_d + exm / denom) * (f32(1.0) / f32(nm + 1))
    w = se / denom
    s_dis = jnp.sum(alpha_dd * ef, axis=0, keepdims=True)
    s_med = jnp.sum(w * ef, axis=0, keepdims=True) * (f32(1.0) / f32(nd))

    sd = jnp.sum(jnp.where(mask_d, dia, f32(0.0)), axis=0, keepdims=True)
    sm = jnp.sum(med, axis=0, keepdims=True)
    se0 = jnp.sum(jnp.where(mask_d, e0_ref[:], f32(0.0)), axis=0, keepdims=True)
    se1 = jnp.sum(jnp.where(mask_m, e1_ref[:], f32(0.0)), axis=0, keepdims=True)

    sum_dia_feat = dot(sd, W1, ((1,), (0,))) + f32(nd) * b1
    sum_med_feat = dot(sm, W1, ((1,), (0,))) + f32(nm) * b1
    u1 = jnp.concatenate([s_dis + f32(nd) * b2, sum_dia_feat], axis=1)
    u2 = jnp.concatenate([s_med + f32(nm) * b2, sum_med_feat], axis=1)
    o1_ref[:] = se0 + dot(u1, Wl, ((1,), (1,)))
    o2_ref[:] = se1 + dot(u2, Wl, ((1,), (1,)))


def _tc_call(nd, nm, dia_g, med_g, e0_g, e1_g, hea_t,
             W1, W2, Wl, b1, b2, att, interpret=False):
    return pl.pallas_call(
        functools.partial(_tc_body, nd, nm),
        out_shape=[jax.ShapeDtypeStruct((1, D), jnp.float32)] * 2,
        interpret=interpret,
    )(dia_g, med_g, e0_g, e1_g, hea_t, W1, W2, Wl,
      b1.reshape(1, D), b2.reshape(1, D), att.reshape(1, 2 * D))


def kernel(c_embeddings, m_embeddings, emb0, emb1, W1, b1, W2, b2, att, Wl,
           hyperedge_attr, c_it, medicine_it):
    nd = c_it.shape[0]
    nm = medicine_it.shape[0]
    dpad = _pad_to(nd, 16 * _NW)
    mpad = _pad_to(nm, 16 * _NW)
    cit_p = jnp.concatenate(
        [c_it.astype(jnp.int32), jnp.zeros((dpad - nd,), jnp.int32)])
    mit_p = jnp.concatenate(
        [medicine_it.astype(jnp.int32), jnp.zeros((mpad - nm,), jnp.int32)])
    dia_g, med_g, e0_g, e1_g = _sc_gather(
        c_embeddings.T, m_embeddings.T, emb0.T, emb1.T, cit_p, mit_p, nm)
    o1, o2 = _tc_call(nd, nm, dia_g, med_g, e0_g, e1_g, hyperedge_attr.T,
                      W1, W2, Wl, b1, b2, att)
    return (o1.reshape(1, 1, D), o2.reshape(1, 1, D))


# speedup vs baseline: 362.9581x; 1.0802x over previous
import functools

import jax
import jax.numpy as jnp
from jax import lax
from jax.experimental import pallas as pl
from jax.experimental.pallas import tpu as pltpu
from jax.experimental.pallas import tpu_sc as plsc

D = 64
LANES = 128
NBUF = 6

try:
    _info = plsc.get_sparse_core_info()
    _NC, _NS = _info.num_cores, _info.num_subcores
except Exception:
    _NC, _NS = 2, 16
_NW = _NC * _NS


def _pad_to(n: int, mult: int) -> int:
    return ((n + mult - 1) // mult) * mult


def _sc_gather(cemb_t, memb_t, emb0_t, emb1_t, cit_p, mit_p, nm):
    dpad = cit_p.shape[0]
    mpad = mit_p.shape[0]
    bd = dpad // _NW
    bm = mpad // _NW
    mesh = plsc.VectorSubcoreMesh(core_axis_name="c", subcore_axis_name="s")

    @functools.partial(
        pl.kernel,
        mesh=mesh,
        compiler_params=pltpu.CompilerParams(needs_layout_passes=False),
        out_type=[
            jax.ShapeDtypeStruct((dpad, D), jnp.float32),
            jax.ShapeDtypeStruct((mpad, D), jnp.float32),
            jax.ShapeDtypeStruct((dpad, D), jnp.float32),
            jax.ShapeDtypeStruct((mpad, D), jnp.float32),
        ],
        scratch_types=[
            pltpu.VMEM((bd,), jnp.int32),
            pltpu.VMEM((bm,), jnp.int32),
            pltpu.VMEM((bd, D), jnp.float32),
            pltpu.VMEM((bm, D), jnp.float32),
            pltpu.VMEM((bd, D), jnp.float32),
            pltpu.VMEM((bm, D), jnp.float32),
            [pltpu.VMEM((D, LANES), jnp.float32) for _ in range(NBUF)],
            [pltpu.SemaphoreType.DMA for _ in range(NBUF)],
        ],
    )
    def k(cemb_h, memb_h, e0_h, e1_h, cit_h, mit_h,
          dia_o, med_o, g0_o, g1_o,
          idxd_v, idxm_v, st_dia, st_med, st_e0, st_e1, blks, sems):
        wid = lax.axis_index("s") * _NC + lax.axis_index("c")
        od = wid * bd
        om = wid * bm
        pltpu.sync_copy(cit_h.at[pl.ds(od, bd)], idxd_v)
        pltpu.sync_copy(mit_h.at[pl.ds(om, bm)], idxm_v)
        vecs_d = [idxd_v[pl.ds(16 * t, 16)] for t in range(bd // 16)]
        vecs_m = [idxm_v[pl.ds(16 * t, 16)] for t in range(bm // 16)]

        jobs = []
        for i in range(bd):
            s = vecs_d[i // 16][i % 16]
            jobs.append((cemb_h, st_dia, i, s, None))
            jobs.append((e0_h, st_e0, i, s, None))
        for i in range(bm):
            s = vecs_m[i // 16][i % 16]
            cond = om + i < nm
            jobs.append((memb_h, st_med, i, s, cond))
            jobs.append((e1_h, st_e1, i, s, cond))

        rows_q = [lax.iota(jnp.int32, 16) + 16 * q for q in range(D // 16)]

        def fire(j, slot):
            src, _, _, s, cond = jobs[j]
            start = pl.multiple_of((s // LANES) * LANES, LANES)

            def do():
                pltpu.make_async_copy(
                    src.at[:, pl.ds(start, LANES)], blks[slot],
                    sems[slot]).start()
            if cond is None:
                do()
            else:
                pl.when(cond)(do)

        def drain(j, slot):
            src, stage, row, s, cond = jobs[j]
            lo = s % LANES

            def do():
                pltpu.make_async_copy(
                    src.at[:, pl.ds(0, LANES)], blks[slot],
                    sems[slot]).wait()
                colv = jnp.full((16,), lo, jnp.int32)
                for q in range(D // 16):
                    g = plsc.load_gather(blks[slot], [rows_q[q], colv])
                    stage[row, pl.ds(16 * q, 16)] = g
            if cond is None:
                do()
            else:
                pl.when(cond)(do)

        n = len(jobs)
        for j in range(n + NBUF):
            if j >= NBUF:
                drain(j - NBUF, (j - NBUF) % NBUF)
            if j < n:
                fire(j, j % NBUF)

        pltpu.sync_copy(st_dia, dia_o.at[pl.ds(od, bd)])
        pltpu.sync_copy(st_e0, g0_o.at[pl.ds(od, bd)])
        pltpu.sync_copy(st_med, med_o.at[pl.ds(om, bm)])
        pltpu.sync_copy(st_e1, g1_o.at[pl.ds(om, bm)])

    return k(cemb_t, memb_t, emb0_t, emb1_t, cit_p, mit_p)


def _tc_body(nd, nm, dia_ref, med_ref, e0_ref, e1_ref, hea_ref,
             w1_ref, w2_ref, wl_ref, b1_ref, b2_ref, att_ref,
             o1_ref, o2_ref):
    f32 = jnp.float32
    hi = lax.Precision.HIGHEST

    def dot(a, b, dn):
        return lax.dot_general(a, b, dimension_numbers=(dn, ((), ())),
                               preferred_element_type=f32, precision=hi)

    dia = dia_ref[:]
    hea_t = hea_ref[:]
    W1 = w1_ref[:]
    W2 = w2_ref[:]
    Wl = wl_ref[:]
    b1 = b1_ref[:]
    b2 = b2_ref[:]
    att = att_ref[:]
    att1 = att[:, :D]
    att2 = att[:, D:]

    dpad = dia.shape[0]
    mt = _pad_to(nm, 8)
    mask_d = lax.broadcasted_iota(jnp.int32, (dpad, 1), 0) < nd
    mask_m = lax.broadcasted_iota(jnp.int32, (mt, 1), 0) < nm
    cmask = lax.broadcasted_iota(jnp.int32, (1, mt), 1) < nm

    med = jnp.where(mask_m, med_ref[:mt], f32(0.0))

    xw_d = dot(dia[:nd], W2, ((1,), (0,)))
    xw_m = dot(med, W2, ((1,), (0,)))
    ew = dot(hea_t, W2, ((0,), (0,)))

    a_d = dot(xw_d, att1, ((1,), (1,)))
    b_e = dot(ew, att2, ((1,), (1,)))
    a_m = dot(att1, xw_m, ((1,), (1,)))

    def lrelu(x):
        return jnp.where(x > 0, x, 0.2 * x)

    s = jnp.where(cmask, lrelu(a_m + b_e), f32(-1e30))
    t = lrelu(a_d + b_e)
    mx = jnp.maximum(jnp.max(s, axis=1, keepdims=True), t)
    e = jnp.exp(s - mx)
    ed = jnp.exp(t - mx)
    se = jnp.sum(e, axis=1, keepdims=True)
    denom = ed + se + f32(1e-16)
    alpha_dd = ed / denom
    exm = dot(e, xw_m, ((1,), (0,)))
    ef = (alpha_dd * xw_d + exm / denom) * (f32(1.0) / f32(nm + 1))
    w = se / denom
    s_dis = jnp.sum(alpha_dd * ef, axis=0, keepdims=True)
    s_med = jnp.sum(w * ef, axis=0, keepdims=True) * (f32(1.0) / f32(nd))

    sd = jnp.sum(jnp.where(mask_d, dia, f32(0.0)), axis=0, keepdims=True)
    sm = jnp.sum(med, axis=0, keepdims=True)
    se0 = jnp.sum(jnp.where(mask_d, e0_ref[:], f32(0.0)), axis=0, keepdims=True)
    se1 = jnp.sum(jnp.where(mask_m, e1_ref[:mt], f32(0.0)), axis=0, keepdims=True)

    sum_dia_feat = dot(sd, W1, ((1,), (0,))) + f32(nd) * b1
    sum_med_feat = dot(sm, W1, ((1,), (0,))) + f32(nm) * b1
    u1 = jnp.concatenate([s_dis + f32(nd) * b2, sum_dia_feat], axis=1)
    u2 = jnp.concatenate([s_med + f32(nm) * b2, sum_med_feat], axis=1)
    o1_ref[:] = se0 + dot(u1, Wl, ((1,), (1,)))
    o2_ref[:] = se1 + dot(u2, Wl, ((1,), (1,)))


def _tc_call(nd, nm, dia_g, med_g, e0_g, e1_g, hea_t,
             W1, W2, Wl, b1, b2, att, interpret=False):
    return pl.pallas_call(
        functools.partial(_tc_body, nd, nm),
        out_shape=[jax.ShapeDtypeStruct((1, D), jnp.float32)] * 2,
        interpret=interpret,
    )(dia_g, med_g, e0_g, e1_g, hea_t, W1, W2, Wl,
      b1.reshape(1, D), b2.reshape(1, D), att.reshape(1, 2 * D))


def kernel(c_embeddings, m_embeddings, emb0, emb1, W1, b1, W2, b2, att, Wl,
           hyperedge_attr, c_it, medicine_it):
    nd = c_it.shape[0]
    nm = medicine_it.shape[0]
    dpad = _pad_to(nd, 16 * _NW)
    mpad = _pad_to(nm, 16 * _NW)
    cit_p = jnp.concatenate(
        [c_it.astype(jnp.int32), jnp.zeros((dpad - nd,), jnp.int32)])
    mit_p = jnp.concatenate(
        [medicine_it.astype(jnp.int32), jnp.zeros((mpad - nm,), jnp.int32)])
    dia_g, med_g, e0_g, e1_g = _sc_gather(
        c_embeddings.T, m_embeddings.T, emb0.T, emb1.T, cit_p, mit_p, nm)
    o1, o2 = _tc_call(nd, nm, dia_g, med_g, e0_g, e1_g, hyperedge_attr.T,
                      W1, W2, Wl, b1, b2, att)
    return (o1.reshape(1, 1, D), o2.reshape(1, 1, D))


# speedup vs baseline: 394.5458x; 1.0870x over previous
import functools

import jax
import jax.numpy as jnp
from jax import lax
from jax.experimental import pallas as pl
from jax.experimental.pallas import tpu as pltpu
from jax.experimental.pallas import tpu_sc as plsc

D = 64
LANES = 128
NBUF = 6

try:
    _info = plsc.get_sparse_core_info()
    _NC, _NS = _info.num_cores, _info.num_subcores
except Exception:
    _NC, _NS = 2, 16
_NW = _NC * _NS


def _pad_to(n: int, mult: int) -> int:
    return ((n + mult - 1) // mult) * mult


def _sc_gather(cemb_t, memb_t, emb0_t, emb1_t, cit_p, mit_p, nm):
    dpad = cit_p.shape[0]
    mpad = mit_p.shape[0]
    bd = dpad // _NW
    bm = mpad // _NW
    mesh = plsc.VectorSubcoreMesh(core_axis_name="c", subcore_axis_name="s")

    @functools.partial(
        pl.kernel,
        mesh=mesh,
        compiler_params=pltpu.CompilerParams(needs_layout_passes=False),
        out_type=[
            jax.ShapeDtypeStruct((dpad, D), jnp.float32),
            jax.ShapeDtypeStruct((mpad, D), jnp.float32),
            jax.ShapeDtypeStruct((dpad, D), jnp.float32),
            jax.ShapeDtypeStruct((mpad, D), jnp.float32),
        ],
        scratch_types=[
            pltpu.VMEM((bd,), jnp.int32),
            pltpu.VMEM((bm,), jnp.int32),
            pltpu.VMEM((bd, D), jnp.float32),
            pltpu.VMEM((bm, D), jnp.float32),
            pltpu.VMEM((bd, D), jnp.float32),
            pltpu.VMEM((bm, D), jnp.float32),
            [pltpu.VMEM((D, LANES), jnp.float32) for _ in range(NBUF)],
            [pltpu.SemaphoreType.DMA for _ in range(NBUF)],
        ],
    )
    def k(cemb_h, memb_h, e0_h, e1_h, cit_h, mit_h,
          dia_o, med_o, g0_o, g1_o,
          idxd_v, idxm_v, st_dia, st_med, st_e0, st_e1, blks, sems):
        wid = lax.axis_index("s") * _NC + lax.axis_index("c")
        od = wid * bd
        om = wid * bm
        pltpu.sync_copy(cit_h.at[pl.ds(od, bd)], idxd_v)
        pltpu.sync_copy(mit_h.at[pl.ds(om, bm)], idxm_v)
        vecs_d = [idxd_v[pl.ds(16 * t, 16)] for t in range(bd // 16)]
        vecs_m = [idxm_v[pl.ds(16 * t, 16)] for t in range(bm // 16)]

        jobs = []
        for i in range(bd):
            s = vecs_d[i // 16][i % 16]
            jobs.append((cemb_h, st_dia, i, s, None))
            jobs.append((e0_h, st_e0, i, s, None))
        for i in range(bm):
            s = vecs_m[i // 16][i % 16]
            cond = om + i < nm
            jobs.append((memb_h, st_med, i, s, cond))
            jobs.append((e1_h, st_e1, i, s, cond))

        rows_q = [lax.iota(jnp.int32, 16) + 16 * q for q in range(D // 16)]

        def fire(j, slot):
            src, _, _, s, cond = jobs[j]
            start = pl.multiple_of((s // LANES) * LANES, LANES)

            def do():
                pltpu.make_async_copy(
                    src.at[:, pl.ds(start, LANES)], blks[slot],
                    sems[slot]).start()
            if cond is None:
                do()
            else:
                pl.when(cond)(do)

        def drain(j, slot):
            src, stage, row, s, cond = jobs[j]
            lo = s % LANES

            def do():
                pltpu.make_async_copy(
                    src.at[:, pl.ds(0, LANES)], blks[slot],
                    sems[slot]).wait()
                colv = jnp.full((16,), lo, jnp.int32)
                for q in range(D // 16):
                    g = plsc.load_gather(blks[slot], [rows_q[q], colv])
                    stage[row, pl.ds(16 * q, 16)] = g
            if cond is None:
                do()
            else:
                pl.when(cond)(do)

        n = len(jobs)
        for j in range(n + NBUF):
            if j >= NBUF:
                drain(j - NBUF, (j - NBUF) % NBUF)
            if j < n:
                fire(j, j % NBUF)

        pltpu.sync_copy(st_dia, dia_o.at[pl.ds(od, bd)])
        pltpu.sync_copy(st_e0, g0_o.at[pl.ds(od, bd)])
        pltpu.sync_copy(st_med, med_o.at[pl.ds(om, bm)])
        pltpu.sync_copy(st_e1, g1_o.at[pl.ds(om, bm)])

    return k(cemb_t, memb_t, emb0_t, emb1_t, cit_p, mit_p)


def _tc_body(nd, nm, dia_ref, med_ref, e0_ref, e1_ref, hea_ref,
             w1_ref, w2_ref, wl_ref, b1_ref, b2_ref, att_ref,
             o1_ref, o2_ref):
    f32 = jnp.float32
    hi = lax.Precision.DEFAULT

    def dot(a, b, dn):
        return lax.dot_general(a, b, dimension_numbers=(dn, ((), ())),
                               preferred_element_type=f32, precision=hi)

    dia = dia_ref[:]
    hea_t = hea_ref[:]
    W1 = w1_ref[:]
    W2 = w2_ref[:]
    Wl = wl_ref[:]
    b1 = b1_ref[:]
    b2 = b2_ref[:]
    att = att_ref[:]
    att1 = att[:, :D]
    att2 = att[:, D:]

    dpad = dia.shape[0]
    mt = _pad_to(nm, 8)
    mask_d = lax.broadcasted_iota(jnp.int32, (dpad, 1), 0) < nd
    mask_m = lax.broadcasted_iota(jnp.int32, (mt, 1), 0) < nm
    cmask = lax.broadcasted_iota(jnp.int32, (1, mt), 1) < nm

    med = jnp.where(mask_m, med_ref[:mt], f32(0.0))

    xw_d = dot(dia[:nd], W2, ((1,), (0,)))
    xw_m = dot(med, W2, ((1,), (0,)))
    ew = dot(hea_t, W2, ((0,), (0,)))

    a_d = dot(xw_d, att1, ((1,), (1,)))
    b_e = dot(ew, att2, ((1,), (1,)))
    a_m = dot(att1, xw_m, ((1,), (1,)))

    def lrelu(x):
        return jnp.where(x > 0, x, 0.2 * x)

    s = jnp.where(cmask, lrelu(a_m + b_e), f32(-1e30))
    t = lrelu(a_d + b_e)
    mx = jnp.maximum(jnp.max(s, axis=1, keepdims=True), t)
    e = jnp.exp(s - mx)
    ed = jnp.exp(t - mx)
    se = jnp.sum(e, axis=1, keepdims=True)
    denom = ed + se + f32(1e-16)
    alpha_dd = ed / denom
    exm = dot(e, xw_m, ((1,), (0,)))
    ef = (alpha_dd * xw_d + exm / denom) * (f32(1.0) / f32(nm + 1))
    w = se / denom
    s_dis = jnp.sum(alpha_dd * ef, axis=0, keepdims=True)
    s_med = jnp.sum(w * ef, axis=0, keepdims=True) * (f32(1.0) / f32(nd))

    sd = jnp.sum(jnp.where(mask_d, dia, f32(0.0)), axis=0, keepdims=True)
    sm = jnp.sum(med, axis=0, keepdims=True)
    se0 = jnp.sum(jnp.where(mask_d, e0_ref[:], f32(0.0)), axis=0, keepdims=True)
    se1 = jnp.sum(jnp.where(mask_m, e1_ref[:mt], f32(0.0)), axis=0, keepdims=True)

    sum_dia_feat = dot(sd, W1, ((1,), (0,))) + f32(nd) * b1
    sum_med_feat = dot(sm, W1, ((1,), (0,))) + f32(nm) * b1
    u1 = jnp.concatenate([s_dis + f32(nd) * b2, sum_dia_feat], axis=1)
    u2 = jnp.concatenate([s_med + f32(nm) * b2, sum_med_feat], axis=1)
    o1_ref[:] = se0 + dot(u1, Wl, ((1,), (1,)))
    o2_ref[:] = se1 + dot(u2, Wl, ((1,), (1,)))


def _tc_call(nd, nm, dia_g, med_g, e0_g, e1_g, hea_t,
             W1, W2, Wl, b1, b2, att, interpret=False):
    return pl.pallas_call(
        functools.partial(_tc_body, nd, nm),
        out_shape=[jax.ShapeDtypeStruct((1, D), jnp.float32)] * 2,
        interpret=interpret,
    )(dia_g, med_g, e0_g, e1_g, hea_t, W1, W2, Wl,
      b1.reshape(1, D), b2.reshape(1, D), att.reshape(1, 2 * D))


def kernel(c_embeddings, m_embeddings, emb0, emb1, W1, b1, W2, b2, att, Wl,
           hyperedge_attr, c_it, medicine_it):
    nd = c_it.shape[0]
    nm = medicine_it.shape[0]
    dpad = _pad_to(nd, 16 * _NW)
    mpad = _pad_to(nm, 16 * _NW)
    cit_p = jnp.concatenate(
        [c_it.astype(jnp.int32), jnp.zeros((dpad - nd,), jnp.int32)])
    mit_p = jnp.concatenate(
        [medicine_it.astype(jnp.int32), jnp.zeros((mpad - nm,), jnp.int32)])
    dia_g, med_g, e0_g, e1_g = _sc_gather(
        c_embeddings.T, m_embeddings.T, emb0.T, emb1.T, cit_p, mit_p, nm)
    o1, o2 = _tc_call(nd, nm, dia_g, med_g, e0_g, e1_g, hyperedge_attr.T,
                      W1, W2, Wl, b1, b2, att)
    return (o1.reshape(1, 1, D), o2.reshape(1, 1, D))


# speedup vs baseline: 411.4839x; 1.0429x over previous
import functools

import jax
import jax.numpy as jnp
from jax import lax
from jax.experimental import pallas as pl
from jax.experimental.pallas import tpu as pltpu
from jax.experimental.pallas import tpu_sc as plsc

D = 64
LANES = 128
NBUF = 6

try:
    _info = plsc.get_sparse_core_info()
    _NC, _NS = _info.num_cores, _info.num_subcores
except Exception:
    _NC, _NS = 2, 16
_NW = _NC * _NS


def _pad_to(n: int, mult: int) -> int:
    return ((n + mult - 1) // mult) * mult


def _sc_gather(cemb_t, memb_t, emb0_t, emb1_t, cit_p, mit_p, nm):
    dpad = cit_p.shape[0]
    bd = dpad // _NW
    bm = -(-nm // _NW)
    mpad = 16 * _NW
    mesh = plsc.VectorSubcoreMesh(core_axis_name="c", subcore_axis_name="s")

    @functools.partial(
        pl.kernel,
        mesh=mesh,
        compiler_params=pltpu.CompilerParams(needs_layout_passes=False),
        out_type=[
            jax.ShapeDtypeStruct((dpad, D), jnp.float32),
            jax.ShapeDtypeStruct((mpad, D), jnp.float32),
            jax.ShapeDtypeStruct((dpad, D), jnp.float32),
            jax.ShapeDtypeStruct((mpad, D), jnp.float32),
        ],
        scratch_types=[
            pltpu.VMEM((bd,), jnp.int32),
            pltpu.VMEM((mit_p.shape[0],), jnp.int32),
            pltpu.VMEM((bd, D), jnp.float32),
            pltpu.VMEM((16, D), jnp.float32),
            pltpu.VMEM((bd, D), jnp.float32),
            pltpu.VMEM((16, D), jnp.float32),
            [pltpu.VMEM((D, LANES), jnp.float32) for _ in range(NBUF)],
            [pltpu.SemaphoreType.DMA for _ in range(NBUF)],
        ],
    )
    def k(cemb_h, memb_h, e0_h, e1_h, cit_h, mit_h,
          dia_o, med_o, g0_o, g1_o,
          idxd_v, idxm_v, st_dia, st_med, st_e0, st_e1, blks, sems):
        wid = lax.axis_index("s") * _NC + lax.axis_index("c")
        od = wid * bd
        pltpu.sync_copy(cit_h.at[pl.ds(od, bd)], idxd_v)
        pltpu.sync_copy(mit_h, idxm_v)
        vecs_d = [idxd_v[pl.ds(16 * t, 16)] for t in range(bd // 16)]
        vec_m = plsc.load_gather(
            idxm_v, [wid * bm + lax.iota(jnp.int32, 16)])

        jobs = []
        for i in range(bd):
            s = vecs_d[i // 16][i % 16]
            jobs.append((cemb_h, st_dia, i, s, None))
            jobs.append((e0_h, st_e0, i, s, None))
        for i in range(bm):
            s = vec_m[i]
            cond = wid * bm + i < nm
            jobs.append((memb_h, st_med, i, s, cond))
            jobs.append((e1_h, st_e1, i, s, cond))

        rows_q = [lax.iota(jnp.int32, 16) + 16 * q for q in range(D // 16)]

        def fire(j, slot):
            src, _, _, s, cond = jobs[j]
            start = pl.multiple_of((s // LANES) * LANES, LANES)

            def do():
                pltpu.make_async_copy(
                    src.at[:, pl.ds(start, LANES)], blks[slot],
                    sems[slot]).start()
            if cond is None:
                do()
            else:
                pl.when(cond)(do)

        def drain(j, slot):
            src, stage, row, s, cond = jobs[j]
            lo = s % LANES

            def do():
                pltpu.make_async_copy(
                    src.at[:, pl.ds(0, LANES)], blks[slot],
                    sems[slot]).wait()
                colv = jnp.full((16,), lo, jnp.int32)
                for q in range(D // 16):
                    g = plsc.load_gather(blks[slot], [rows_q[q], colv])
                    stage[row, pl.ds(16 * q, 16)] = g
            if cond is None:
                do()
            else:
                pl.when(cond)(do)

        n = len(jobs)
        for j in range(n + NBUF):
            if j >= NBUF:
                drain(j - NBUF, (j - NBUF) % NBUF)
            if j < n:
                fire(j, j % NBUF)

        pltpu.sync_copy(st_dia, dia_o.at[pl.ds(od, bd)])
        pltpu.sync_copy(st_e0, g0_o.at[pl.ds(od, bd)])
        pltpu.sync_copy(st_med, med_o.at[pl.ds(16 * wid, 16)])
        pltpu.sync_copy(st_e1, g1_o.at[pl.ds(16 * wid, 16)])

    return k(cemb_t, memb_t, emb0_t, emb1_t, cit_p, mit_p)


def _tc_body(nd, nm, bm, dia_ref, med_ref, e0_ref, e1_ref, hea_ref,
             w1_ref, w2_ref, wl_ref, b1_ref, b2_ref, att_ref,
             o1_ref, o2_ref):
    f32 = jnp.float32
    hi = lax.Precision.DEFAULT

    def dot(a, b, dn):
        return lax.dot_general(a, b, dimension_numbers=(dn, ((), ())),
                               preferred_element_type=f32, precision=hi)

    dia = dia_ref[:]
    hea_t = hea_ref[:]
    W1 = w1_ref[:]
    W2 = w2_ref[:]
    Wl = wl_ref[:]
    b1 = b1_ref[:]
    b2 = b2_ref[:]
    att = att_ref[:]
    att1 = att[:, :D]
    att2 = att[:, D:]

    dpad = dia.shape[0]
    mpad = med_ref.shape[0]
    mask_d = lax.broadcasted_iota(jnp.int32, (dpad, 1), 0) < nd

    def slot_valid(shape, dim):
        r = lax.broadcasted_iota(jnp.int32, shape, dim)
        sl = r % 16
        return (sl < bm) & ((r // 16) * bm + sl < nm)

    mask_m = slot_valid((mpad, 1), 0)
    cmask = slot_valid((1, mpad), 1)

    med = jnp.where(mask_m, med_ref[:], f32(0.0))

    xw_d = dot(dia[:nd], W2, ((1,), (0,)))
    xw_m = dot(med, W2, ((1,), (0,)))
    ew = dot(hea_t, W2, ((0,), (0,)))

    a_d = dot(xw_d, att1, ((1,), (1,)))
    b_e = dot(ew, att2, ((1,), (1,)))
    a_m = dot(att1, xw_m, ((1,), (1,)))

    def lrelu(x):
        return jnp.where(x > 0, x, 0.2 * x)

    s = jnp.where(cmask, lrelu(a_m + b_e), f32(-1e30))
    t = lrelu(a_d + b_e)
    mx = jnp.maximum(jnp.max(s, axis=1, keepdims=True), t)
    e = jnp.exp(s - mx)
    ed = jnp.exp(t - mx)
    se = jnp.sum(e, axis=1, keepdims=True)
    denom = ed + se + f32(1e-16)
    alpha_dd = ed / denom
    exm = dot(e, xw_m, ((1,), (0,)))
    ef = (alpha_dd * xw_d + exm / denom) * (f32(1.0) / f32(nm + 1))
    w = se / denom
    s_dis = jnp.sum(alpha_dd * ef, axis=0, keepdims=True)
    s_med = jnp.sum(w * ef, axis=0, keepdims=True) * (f32(1.0) / f32(nd))

    sd = jnp.sum(jnp.where(mask_d, dia, f32(0.0)), axis=0, keepdims=True)
    sm = jnp.sum(med, axis=0, keepdims=True)
    se0 = jnp.sum(jnp.where(mask_d, e0_ref[:], f32(0.0)), axis=0, keepdims=True)
    se1 = jnp.sum(jnp.where(mask_m, e1_ref[:], f32(0.0)), axis=0, keepdims=True)

    sum_dia_feat = dot(sd, W1, ((1,), (0,))) + f32(nd) * b1
    sum_med_feat = dot(sm, W1, ((1,), (0,))) + f32(nm) * b1
    u1 = jnp.concatenate([s_dis + f32(nd) * b2, sum_dia_feat], axis=1)
    u2 = jnp.concatenate([s_med + f32(nm) * b2, sum_med_feat], axis=1)
    o1_ref[:] = se0 + dot(u1, Wl, ((1,), (1,)))
    o2_ref[:] = se1 + dot(u2, Wl, ((1,), (1,)))


def _tc_call(nd, nm, bm, dia_g, med_g, e0_g, e1_g, hea_t,
             W1, W2, Wl, b1, b2, att, interpret=False):
    return pl.pallas_call(
        functools.partial(_tc_body, nd, nm, bm),
        out_shape=[jax.ShapeDtypeStruct((1, D), jnp.float32)] * 2,
        interpret=interpret,
    )(dia_g, med_g, e0_g, e1_g, hea_t, W1, W2, Wl,
      b1.reshape(1, D), b2.reshape(1, D), att.reshape(1, 2 * D))


def kernel(c_embeddings, m_embeddings, emb0, emb1, W1, b1, W2, b2, att, Wl,
           hyperedge_attr, c_it, medicine_it):
    nd = c_it.shape[0]
    nm = medicine_it.shape[0]
    dpad = _pad_to(nd, 16 * _NW)
    midx = _pad_to(-(-nm // _NW) * _NW + 16, 16)
    cit_p = jnp.concatenate(
        [c_it.astype(jnp.int32), jnp.zeros((dpad - nd,), jnp.int32)])
    mit_p = jnp.concatenate(
        [medicine_it.astype(jnp.int32), jnp.zeros((midx - nm,), jnp.int32)])
    dia_g, med_g, e0_g, e1_g = _sc_gather(
        c_embeddings.T, m_embeddings.T, emb0.T, emb1.T, cit_p, mit_p, nm)
    o1, o2 = _tc_call(nd, nm, -(-nm // _NW), dia_g, med_g, e0_g, e1_g,
                      hyperedge_attr.T, W1, W2, Wl, b1, b2, att)
    return (o1.reshape(1, 1, D), o2.reshape(1, 1, D))


# speedup vs baseline: 413.9864x; 1.0061x over previous
import functools

import jax
import jax.numpy as jnp
from jax import lax
from jax.experimental import pallas as pl
from jax.experimental.pallas import tpu as pltpu
from jax.experimental.pallas import tpu_sc as plsc

D = 64
LANES = 128
NBUF = 6

try:
    _info = plsc.get_sparse_core_info()
    _NC, _NS = _info.num_cores, _info.num_subcores
except Exception:
    _NC, _NS = 2, 16
_NW = _NC * _NS


def _pad_to(n: int, mult: int) -> int:
    return ((n + mult - 1) // mult) * mult


def _sc_gather(cemb_t, memb_t, emb0_t, emb1_t, cit, mit):
    nd = cit.shape[0]
    nm = mit.shape[0]
    bd = -(-nd // (16 * _NW)) * 16
    dpad = bd * _NW
    full_d = nd // bd
    rem_d = nd % bd
    bm = -(-nm // _NW)
    mpad = 16 * _NW
    midx = _pad_to(bm * _NW + 16, 16)
    mesh = plsc.VectorSubcoreMesh(core_axis_name="c", subcore_axis_name="s")

    @functools.partial(
        pl.kernel,
        mesh=mesh,
        compiler_params=pltpu.CompilerParams(needs_layout_passes=False),
        out_type=[
            jax.ShapeDtypeStruct((dpad, D), jnp.float32),
            jax.ShapeDtypeStruct((mpad, D), jnp.float32),
            jax.ShapeDtypeStruct((dpad, D), jnp.float32),
            jax.ShapeDtypeStruct((mpad, D), jnp.float32),
        ],
        scratch_types=[
            pltpu.VMEM((bd,), jnp.int32),
            pltpu.VMEM((midx,), jnp.int32),
            pltpu.VMEM((bd, D), jnp.float32),
            pltpu.VMEM((16, D), jnp.float32),
            pltpu.VMEM((bd, D), jnp.float32),
            pltpu.VMEM((16, D), jnp.float32),
            [pltpu.VMEM((D, LANES), jnp.float32) for _ in range(NBUF)],
            [pltpu.SemaphoreType.DMA for _ in range(NBUF)],
        ],
    )
    def k(cemb_h, memb_h, e0_h, e1_h, cit_h, mit_h,
          dia_o, med_o, g0_o, g1_o,
          idxd_v, idxm_v, st_dia, st_med, st_e0, st_e1, blks, sems):
        wid = lax.axis_index("s") * _NC + lax.axis_index("c")
        od = wid * bd
        pl.when(wid < full_d)(
            lambda: pltpu.sync_copy(cit_h.at[pl.ds(od, bd)], idxd_v))
        if rem_d:
            pl.when(wid == full_d)(
                lambda: pltpu.sync_copy(
                    cit_h.at[pl.ds(full_d * bd, rem_d)],
                    idxd_v.at[pl.ds(0, rem_d)]))
        pltpu.sync_copy(mit_h, idxm_v.at[pl.ds(0, nm)])
        vecs_d = [idxd_v[pl.ds(16 * t, 16)] for t in range(bd // 16)]
        vec_m = plsc.load_gather(
            idxm_v, [wid * bm + lax.iota(jnp.int32, 16)])

        jobs = []
        for i in range(bd):
            s = vecs_d[i // 16][i % 16]
            cond = None if (i < rem_d or not rem_d) else od + i < nd
            jobs.append((cemb_h, st_dia, i, s, cond))
            jobs.append((e0_h, st_e0, i, s, cond))
        for i in range(bm):
            s = vec_m[i]
            cond = wid * bm + i < nm
            jobs.append((memb_h, st_med, i, s, cond))
            jobs.append((e1_h, st_e1, i, s, cond))

        rows_q = [lax.iota(jnp.int32, 16) + 16 * q for q in range(D // 16)]

        def fire(j, slot):
            src, _, _, s, cond = jobs[j]
            start = pl.multiple_of((s // LANES) * LANES, LANES)

            def do():
                pltpu.make_async_copy(
                    src.at[:, pl.ds(start, LANES)], blks[slot],
                    sems[slot]).start()
            if cond is None:
                do()
            else:
                pl.when(cond)(do)

        def drain(j, slot):
            src, stage, row, s, cond = jobs[j]
            lo = s % LANES

            def do():
                pltpu.make_async_copy(
                    src.at[:, pl.ds(0, LANES)], blks[slot],
                    sems[slot]).wait()
                colv = jnp.full((16,), lo, jnp.int32)
                for q in range(D // 16):
                    g = plsc.load_gather(blks[slot], [rows_q[q], colv])
                    stage[row, pl.ds(16 * q, 16)] = g
            if cond is None:
                do()
            else:
                pl.when(cond)(do)

        n = len(jobs)
        for j in range(n + NBUF):
            if j >= NBUF:
                drain(j - NBUF, (j - NBUF) % NBUF)
            if j < n:
                fire(j, j % NBUF)

        pltpu.sync_copy(st_dia, dia_o.at[pl.ds(od, bd)])
        pltpu.sync_copy(st_e0, g0_o.at[pl.ds(od, bd)])
        pltpu.sync_copy(st_med, med_o.at[pl.ds(16 * wid, 16)])
        pltpu.sync_copy(st_e1, g1_o.at[pl.ds(16 * wid, 16)])

    return k(cemb_t, memb_t, emb0_t, emb1_t, cit, mit)


def _tc_body(nd, nm, bm, dia_ref, med_ref, e0_ref, e1_ref, hea_ref,
             w1_ref, w2_ref, wl_ref, b1_ref, b2_ref, att_ref,
             o1_ref, o2_ref):
    f32 = jnp.float32
    hi = lax.Precision.DEFAULT

    def dot(a, b, dn):
        return lax.dot_general(a, b, dimension_numbers=(dn, ((), ())),
                               preferred_element_type=f32, precision=hi)

    dia = dia_ref[:]
    hea_t = hea_ref[:]
    W1 = w1_ref[:]
    W2 = w2_ref[:]
    Wl = wl_ref[:]
    b1 = b1_ref[:]
    b2 = b2_ref[:]
    att = att_ref[:]
    att1 = att[:, :D]
    att2 = att[:, D:]

    dpad = dia.shape[0]
    mpad = med_ref.shape[0]
    mask_d = lax.broadcasted_iota(jnp.int32, (dpad, 1), 0) < nd

    def slot_valid(shape, dim):
        r = lax.broadcasted_iota(jnp.int32, shape, dim)
        sl = r % 16
        return (sl < bm) & ((r // 16) * bm + sl < nm)

    mask_m = slot_valid((mpad, 1), 0)
    cmask = slot_valid((1, mpad), 1)

    med = jnp.where(mask_m, med_ref[:], f32(0.0))

    xw_d = dot(dia[:nd], W2, ((1,), (0,)))
    xw_m = dot(med, W2, ((1,), (0,)))
    ew = dot(hea_t, W2, ((0,), (0,)))

    a_d = dot(xw_d, att1, ((1,), (1,)))
    b_e = dot(ew, att2, ((1,), (1,)))
    a_m = dot(att1, xw_m, ((1,), (1,)))

    def lrelu(x):
        return jnp.where(x > 0, x, 0.2 * x)

    s = jnp.where(cmask, lrelu(a_m + b_e), f32(-1e30))
    t = lrelu(a_d + b_e)
    mx = jnp.maximum(jnp.max(s, axis=1, keepdims=True), t)
    e = jnp.exp(s - mx)
    ed = jnp.exp(t - mx)
    se = jnp.sum(e, axis=1, keepdims=True)
    denom = ed + se + f32(1e-16)
    alpha_dd = ed / denom
    exm = dot(e, xw_m, ((1,), (0,)))
    ef = (alpha_dd * xw_d + exm / denom) * (f32(1.0) / f32(nm + 1))
    w = se / denom
    s_dis = jnp.sum(alpha_dd * ef, axis=0, keepdims=True)
    s_med = jnp.sum(w * ef, axis=0, keepdims=True) * (f32(1.0) / f32(nd))

    sd = jnp.sum(jnp.where(mask_d, dia, f32(0.0)), axis=0, keepdims=True)
    sm = jnp.sum(med, axis=0, keepdims=True)
    se0 = jnp.sum(jnp.where(mask_d, e0_ref[:], f32(0.0)), axis=0, keepdims=True)
    se1 = jnp.sum(jnp.where(mask_m, e1_ref[:], f32(0.0)), axis=0, keepdims=True)

    sum_dia_feat = dot(sd, W1, ((1,), (0,))) + f32(nd) * b1
    sum_med_feat = dot(sm, W1, ((1,), (0,))) + f32(nm) * b1
    u1 = jnp.concatenate([s_dis + f32(nd) * b2, sum_dia_feat], axis=1)
    u2 = jnp.concatenate([s_med + f32(nm) * b2, sum_med_feat], axis=1)
    o1_ref[:] = se0 + dot(u1, Wl, ((1,), (1,)))
    o2_ref[:] = se1 + dot(u2, Wl, ((1,), (1,)))


def _tc_call(nd, nm, bm, dia_g, med_g, e0_g, e1_g, hea_t,
             W1, W2, Wl, b1, b2, att, interpret=False):
    return pl.pallas_call(
        functools.partial(_tc_body, nd, nm, bm),
        out_shape=[jax.ShapeDtypeStruct((1, D), jnp.float32)] * 2,
        interpret=interpret,
    )(dia_g, med_g, e0_g, e1_g, hea_t, W1, W2, Wl,
      b1.reshape(1, D), b2.reshape(1, D), att.reshape(1, 2 * D))


def kernel(c_embeddings, m_embeddings, emb0, emb1, W1, b1, W2, b2, att, Wl,
           hyperedge_attr, c_it, medicine_it):
    nd = c_it.shape[0]
    nm = medicine_it.shape[0]
    dia_g, med_g, e0_g, e1_g = _sc_gather(
        c_embeddings.T, m_embeddings.T, emb0.T, emb1.T,
        c_it.astype(jnp.int32), medicine_it.astype(jnp.int32))
    o1, o2 = _tc_call(nd, nm, -(-nm // _NW), dia_g, med_g, e0_g, e1_g,
                      hyperedge_attr.T, W1, W2, Wl, b1, b2, att)
    return (o1.reshape(1, 1, D), o2.reshape(1, 1, D))
